# baseline (speedup 1.0000x reference)
import functools

import jax
import jax.numpy as jnp
from jax import lax
from jax.experimental import pallas as pl
from jax.experimental.pallas import tpu as pltpu
from jax.experimental.pallas import tpu_sc as plsc

N = 524288
C = 19
B = 2048
TWO_B = 2 * B
NC = 2
NS = 16
NW = NC * NS
RT = N // NW
P = 1024
NCHK = RT // P
TBL = C * TWO_B
L = 16


def _sc_hist_body(xt_hbm, t_hbm, out_hbm,
                  hist, xb0, xb1, tb0, tb1, sx0, sx1, st0, st1):
    w = lax.axis_index("s") * NC + lax.axis_index("c")
    zero16 = jnp.zeros((L,), jnp.float32)
    ones16 = jnp.ones((L,), jnp.float32)

    def _zero(i, carry):
        for u in range(8):
            hist[pl.ds(i * (8 * L) + u * L, L)] = zero16
        return carry

    lax.fori_loop(0, TBL // (8 * L), _zero, 0)

    p_base = w * RT
    bufs = ((xb0, tb0, sx0, st0), (xb1, tb1, sx1, st1))

    def issue(ch, buf):
        xb, tb, sx, st = buf
        p0 = p_base + ch * P
        hx = pltpu.async_copy(xt_hbm.at[:, pl.ds(p0, P)], xb, sx)
        ht = pltpu.async_copy(t_hbm.at[pl.ds(p0, P)], tb, st)
        return hx, ht

    def wait(ch, buf):
        xb, tb, sx, st = buf
        p0 = p_base + ch * P
        pltpu.make_async_copy(xt_hbm.at[:, pl.ds(p0, P)], xb, sx).wait()
        pltpu.make_async_copy(t_hbm.at[pl.ds(p0, P)], tb, st).wait()

    def process(buf):
        xb, tb = buf[0], buf[1]

        def group_body(g, carry):
            gb = g * L
            t16 = tb[pl.ds(gb, L)]
            es = [jnp.exp(xb[c, pl.ds(gb, L)]) for c in range(C)]
            vals = list(es)
            while len(vals) > 1:
                nxt = [vals[i] + vals[i + 1] for i in range(0, len(vals) - 1, 2)]
                if len(vals) % 2:
                    nxt.append(vals[-1])
                vals = nxt
            inv_b = jnp.float32(B) / vals[0]
            for c in range(C):
                s = es[c] * inv_b
                bk = jnp.minimum(s.astype(jnp.int32), B - 1)
                idx = jnp.where(t16 == c,
                                (bk ^ (B - 1)) + (c * TWO_B + B),
                                bk + c * TWO_B)
                plsc.addupdate_scatter(hist, [idx], ones16)
            return carry

        lax.fori_loop(0, P // L, group_body, 0, unroll=4)

    issue(0, bufs[0])
    issue(1, bufs[1])

    def pair_body(k, carry):
        ch0 = 2 * k
        wait(ch0, bufs[0])
        process(bufs[0])

        @pl.when(ch0 + 2 < NCHK)
        def _():
            issue(ch0 + 2, bufs[0])

        wait(ch0 + 1, bufs[1])
        process(bufs[1])

        @pl.when(ch0 + 3 < NCHK)
        def _():
            issue(ch0 + 3, bufs[1])

        return carry

    lax.fori_loop(0, NCHK // 2, pair_body, 0)
    pltpu.sync_copy(hist, out_hbm.at[pl.ds(w * TBL, TBL)])


def _sc_hist(xt, t_flat):
    mesh = plsc.VectorSubcoreMesh(
        core_axis_name="c", subcore_axis_name="s",
        num_cores=NC, num_subcores=NS)
    kern = functools.partial(
        pl.kernel,
        out_type=jax.ShapeDtypeStruct((NW * TBL,), jnp.float32),
        mesh=mesh,
        scratch_types=[
            pltpu.VMEM((TBL,), jnp.float32),
            pltpu.VMEM((C, P), jnp.float32),
            pltpu.VMEM((C, P), jnp.float32),
            pltpu.VMEM((P,), jnp.int32),
            pltpu.VMEM((P,), jnp.int32),
            pltpu.SemaphoreType.DMA,
            pltpu.SemaphoreType.DMA,
            pltpu.SemaphoreType.DMA,
            pltpu.SemaphoreType.DMA,
        ],
        compiler_params=pltpu.CompilerParams(needs_layout_passes=False),
    )(_sc_hist_body)
    return kern(xt, t_flat)


def _fin_body(h_ref, o_ref):
    s = h_ref[pl.ds(0, TBL)]
    for wk in range(1, NW):
        s = s + h_ref[pl.ds(wk * TBL, TBL)]
    cnt_n = jnp.stack([s[c * TWO_B:c * TWO_B + B] for c in range(C)])
    cnt_p = jnp.stack([s[c * TWO_B + B:(c + 1) * TWO_B] for c in range(C)])
    cnt_a = cnt_n + cnt_p
    row = lax.broadcasted_iota(jnp.int32, (B, B), 0)
    col = lax.broadcasted_iota(jnp.int32, (B, B), 1)
    m = (row > col).astype(jnp.float32)
    dn = (((1,), (0,)), ((), ()))
    above_a = lax.dot_general(cnt_a, m, dn,
                              precision=lax.Precision.HIGHEST,
                              preferred_element_type=jnp.float32)
    above_p = lax.dot_general(cnt_p, m, dn,
                              precision=lax.Precision.HIGHEST,
                              preferred_element_type=jnp.float32)
    tot_a = jnp.sum(cnt_a, axis=-1, keepdims=True)
    tot_p = jnp.sum(cnt_p, axis=-1, keepdims=True)
    r_mid = above_a + 0.5 * cnt_a
    k_mid = above_p + 0.5 * cnt_p
    denom = tot_p + r_mid - k_mid
    j = 1.0 - (tot_p - k_mid) / denom
    j = jnp.where((above_a + cnt_a) > 0.0, j, 0.0)
    o_ref[0, 0] = jnp.sum(j) * (1.0 / (B * C))


def _finish(hist):
    return pl.pallas_call(
        _fin_body,
        out_shape=jax.ShapeDtypeStruct((1, 1), jnp.float32),
        out_specs=pl.BlockSpec(memory_space=pltpu.SMEM),
    )(hist)


def kernel(inputs, targets):
    xt = inputs.T
    t = targets.astype(jnp.int32)
    hist = _sc_hist(xt, t)
    out = _finish(hist)
    return out[0, 0]


# speedup vs baseline: 1.0093x; 1.0093x over previous
import functools

import jax
import jax.numpy as jnp
from jax import lax
from jax.experimental import pallas as pl
from jax.experimental.pallas import tpu as pltpu
from jax.experimental.pallas import tpu_sc as plsc

N = 524288
C = 19
B = 2048
TWO_B = 2 * B
NC = 2
NS = 16
NW = NC * NS
RT = N // NW
P = 1024
NCHK = RT // P
TBL = C * TWO_B
L = 16


def _sc_hist_body(xt_hbm, t_hbm, out_hbm,
                  hist, xb0, xb1, tb0, tb1, sx0, sx1, st0, st1):
    w = lax.axis_index("s") * NC + lax.axis_index("c")
    zero16 = jnp.zeros((L,), jnp.float32)
    ones16 = jnp.ones((L,), jnp.float32)

    def _zero(i, carry):
        for u in range(8):
            hist[pl.ds(i * (8 * L) + u * L, L)] = zero16
        return carry

    lax.fori_loop(0, TBL // (8 * L), _zero, 0)

    p_base = w * RT
    bufs = ((xb0, tb0, sx0, st0), (xb1, tb1, sx1, st1))

    def issue(ch, buf):
        xb, tb, sx, st = buf
        p0 = p_base + ch * P
        hx = pltpu.async_copy(xt_hbm.at[:, pl.ds(p0, P)], xb, sx)
        ht = pltpu.async_copy(t_hbm.at[pl.ds(p0, P)], tb, st)
        return hx, ht

    def wait(ch, buf):
        xb, tb, sx, st = buf
        p0 = p_base + ch * P
        pltpu.make_async_copy(xt_hbm.at[:, pl.ds(p0, P)], xb, sx).wait()
        pltpu.make_async_copy(t_hbm.at[pl.ds(p0, P)], tb, st).wait()

    def process(buf):
        xb, tb = buf[0], buf[1]

        def group_body(g, carry):
            gb = g * L
            t16 = tb[pl.ds(gb, L)]
            es = [jnp.exp(xb[c, pl.ds(gb, L)]) for c in range(C)]
            vals = list(es)
            while len(vals) > 1:
                nxt = [vals[i] + vals[i + 1] for i in range(0, len(vals) - 1, 2)]
                if len(vals) % 2:
                    nxt.append(vals[-1])
                vals = nxt
            inv_b = jnp.float32(B) / vals[0]
            for c in range(C):
                s = es[c] * inv_b
                bk = jnp.minimum(s.astype(jnp.int32), B - 1)
                idx = jnp.where(t16 == c,
                                (bk ^ (B - 1)) + (c * TWO_B + B),
                                bk + c * TWO_B)
                plsc.addupdate_scatter(hist, [idx], ones16)
            return carry

        lax.fori_loop(0, P // L, group_body, 0, unroll=2)

    issue(0, bufs[0])
    issue(1, bufs[1])

    def pair_body(k, carry):
        ch0 = 2 * k
        wait(ch0, bufs[0])
        process(bufs[0])

        @pl.when(ch0 + 2 < NCHK)
        def _():
            issue(ch0 + 2, bufs[0])

        wait(ch0 + 1, bufs[1])
        process(bufs[1])

        @pl.when(ch0 + 3 < NCHK)
        def _():
            issue(ch0 + 3, bufs[1])

        return carry

    lax.fori_loop(0, NCHK // 2, pair_body, 0)
    pltpu.sync_copy(hist, out_hbm.at[pl.ds(w * TBL, TBL)])


def _sc_hist(xt, t_flat):
    mesh = plsc.VectorSubcoreMesh(
        core_axis_name="c", subcore_axis_name="s",
        num_cores=NC, num_subcores=NS)
    kern = functools.partial(
        pl.kernel,
        out_type=jax.ShapeDtypeStruct((NW * TBL,), jnp.float32),
        mesh=mesh,
        scratch_types=[
            pltpu.VMEM((TBL,), jnp.float32),
            pltpu.VMEM((C, P), jnp.float32),
            pltpu.VMEM((C, P), jnp.float32),
            pltpu.VMEM((P,), jnp.int32),
            pltpu.VMEM((P,), jnp.int32),
            pltpu.SemaphoreType.DMA,
            pltpu.SemaphoreType.DMA,
            pltpu.SemaphoreType.DMA,
            pltpu.SemaphoreType.DMA,
        ],
        compiler_params=pltpu.CompilerParams(needs_layout_passes=False),
    )(_sc_hist_body)
    return kern(xt, t_flat)


def _fin_body(h_ref, o_ref):
    s = h_ref[pl.ds(0, TBL)]
    for wk in range(1, NW):
        s = s + h_ref[pl.ds(wk * TBL, TBL)]
    cnt_n = jnp.stack([s[c * TWO_B:c * TWO_B + B] for c in range(C)])
    cnt_p = jnp.stack([s[c * TWO_B + B:(c + 1) * TWO_B] for c in range(C)])
    cnt_a = cnt_n + cnt_p
    row = lax.broadcasted_iota(jnp.int32, (B, B), 0)
    col = lax.broadcasted_iota(jnp.int32, (B, B), 1)
    m = (row > col).astype(jnp.float32)
    dn = (((1,), (0,)), ((), ()))
    above_a = lax.dot_general(cnt_a, m, dn,
                              precision=lax.Precision.HIGHEST,
                              preferred_element_type=jnp.float32)
    above_p = lax.dot_general(cnt_p, m, dn,
                              precision=lax.Precision.HIGHEST,
                              preferred_element_type=jnp.float32)
    tot_a = jnp.sum(cnt_a, axis=-1, keepdims=True)
    tot_p = jnp.sum(cnt_p, axis=-1, keepdims=True)
    r_mid = above_a + 0.5 * cnt_a
    k_mid = above_p + 0.5 * cnt_p
    denom = tot_p + r_mid - k_mid
    j = 1.0 - (tot_p - k_mid) / denom
    j = jnp.where((above_a + cnt_a) > 0.0, j, 0.0)
    o_ref[0, 0] = jnp.sum(j) * (1.0 / (B * C))


def _finish(hist):
    return pl.pallas_call(
        _fin_body,
        out_shape=jax.ShapeDtypeStruct((1, 1), jnp.float32),
        out_specs=pl.BlockSpec(memory_space=pltpu.SMEM),
    )(hist)


def kernel(inputs, targets):
    xt = inputs.T
    t = targets.astype(jnp.int32)
    hist = _sc_hist(xt, t)
    out = _finish(hist)
    return out[0, 0]


# speedup vs baseline: 1.0865x; 1.0765x over previous
import functools

import jax
import jax.numpy as jnp
from jax import lax
from jax.experimental import pallas as pl
from jax.experimental.pallas import tpu as pltpu
from jax.experimental.pallas import tpu_sc as plsc

N = 524288
C = 19
B = 1024
TWO_B = 2 * B
NC = 2
NS = 16
NW = NC * NS
RT = N // NW
P = 1024
NCHK = RT // P
TBL = C * TWO_B
L = 16


def _sc_hist_body(xt_hbm, t_hbm, out_hbm,
                  hist, xb0, xb1, tb0, tb1, sx0, sx1, st0, st1):
    w = lax.axis_index("s") * NC + lax.axis_index("c")
    zero16 = jnp.zeros((L,), jnp.float32)
    ones16 = jnp.ones((L,), jnp.float32)

    def _zero(i, carry):
        for u in range(8):
            hist[pl.ds(i * (8 * L) + u * L, L)] = zero16
        return carry

    lax.fori_loop(0, TBL // (8 * L), _zero, 0)

    p_base = w * RT
    bufs = ((xb0, tb0, sx0, st0), (xb1, tb1, sx1, st1))

    def issue(ch, buf):
        xb, tb, sx, st = buf
        p0 = p_base + ch * P
        hx = pltpu.async_copy(xt_hbm.at[:, pl.ds(p0, P)], xb, sx)
        ht = pltpu.async_copy(t_hbm.at[pl.ds(p0, P)], tb, st)
        return hx, ht

    def wait(ch, buf):
        xb, tb, sx, st = buf
        p0 = p_base + ch * P
        pltpu.make_async_copy(xt_hbm.at[:, pl.ds(p0, P)], xb, sx).wait()
        pltpu.make_async_copy(t_hbm.at[pl.ds(p0, P)], tb, st).wait()

    def process(buf):
        xb, tb = buf[0], buf[1]

        def group_body(g, carry):
            gb = g * L
            t16 = tb[pl.ds(gb, L)]
            es = [jnp.exp(xb[c, pl.ds(gb, L)]) for c in range(C)]
            vals = list(es)
            while len(vals) > 1:
                nxt = [vals[i] + vals[i + 1] for i in range(0, len(vals) - 1, 2)]
                if len(vals) % 2:
                    nxt.append(vals[-1])
                vals = nxt
            inv_b = jnp.float32(B) / vals[0]
            for c in range(C):
                s = es[c] * inv_b
                bk = jnp.minimum(s.astype(jnp.int32), B - 1)
                idx = jnp.where(t16 == c,
                                (bk ^ (B - 1)) + (c * TWO_B + B),
                                bk + c * TWO_B)
                plsc.addupdate_scatter(hist, [idx], ones16)
            return carry

        lax.fori_loop(0, P // L, group_body, 0, unroll=2)

    issue(0, bufs[0])
    issue(1, bufs[1])

    def pair_body(k, carry):
        ch0 = 2 * k
        wait(ch0, bufs[0])
        process(bufs[0])

        @pl.when(ch0 + 2 < NCHK)
        def _():
            issue(ch0 + 2, bufs[0])

        wait(ch0 + 1, bufs[1])
        process(bufs[1])

        @pl.when(ch0 + 3 < NCHK)
        def _():
            issue(ch0 + 3, bufs[1])

        return carry

    lax.fori_loop(0, NCHK // 2, pair_body, 0)
    pltpu.sync_copy(hist, out_hbm.at[pl.ds(w * TBL, TBL)])


def _sc_hist(xt, t_flat):
    mesh = plsc.VectorSubcoreMesh(
        core_axis_name="c", subcore_axis_name="s",
        num_cores=NC, num_subcores=NS)
    kern = functools.partial(
        pl.kernel,
        out_type=jax.ShapeDtypeStruct((NW * TBL,), jnp.float32),
        mesh=mesh,
        scratch_types=[
            pltpu.VMEM((TBL,), jnp.float32),
            pltpu.VMEM((C, P), jnp.float32),
            pltpu.VMEM((C, P), jnp.float32),
            pltpu.VMEM((P,), jnp.int32),
            pltpu.VMEM((P,), jnp.int32),
            pltpu.SemaphoreType.DMA,
            pltpu.SemaphoreType.DMA,
            pltpu.SemaphoreType.DMA,
            pltpu.SemaphoreType.DMA,
        ],
        compiler_params=pltpu.CompilerParams(needs_layout_passes=False),
    )(_sc_hist_body)
    return kern(xt, t_flat)


def _fin_body(h_ref, o_ref):
    s = h_ref[pl.ds(0, TBL)]
    for wk in range(1, NW):
        s = s + h_ref[pl.ds(wk * TBL, TBL)]
    cnt_n = jnp.stack([s[c * TWO_B:c * TWO_B + B] for c in range(C)])
    cnt_p = jnp.stack([s[c * TWO_B + B:(c + 1) * TWO_B] for c in range(C)])
    cnt_a = cnt_n + cnt_p
    row = lax.broadcasted_iota(jnp.int32, (B, B), 0)
    col = lax.broadcasted_iota(jnp.int32, (B, B), 1)
    m = (row > col).astype(jnp.float32)
    dn = (((1,), (0,)), ((), ()))
    above_a = lax.dot_general(cnt_a, m, dn,
                              precision=lax.Precision.HIGHEST,
                              preferred_element_type=jnp.float32)
    above_p = lax.dot_general(cnt_p, m, dn,
                              precision=lax.Precision.HIGHEST,
                              preferred_element_type=jnp.float32)
    tot_a = jnp.sum(cnt_a, axis=-1, keepdims=True)
    tot_p = jnp.sum(cnt_p, axis=-1, keepdims=True)
    r_mid = above_a + 0.5 * cnt_a
    k_mid = above_p + 0.5 * cnt_p
    denom = tot_p + r_mid - k_mid
    j = 1.0 - (tot_p - k_mid) / denom
    j = jnp.where((above_a + cnt_a) > 0.0, j, 0.0)
    o_ref[0, 0] = jnp.sum(j) * (1.0 / (B * C))


def _finish(hist):
    return pl.pallas_call(
        _fin_body,
        out_shape=jax.ShapeDtypeStruct((1, 1), jnp.float32),
        out_specs=pl.BlockSpec(memory_space=pltpu.SMEM),
    )(hist)


def kernel(inputs, targets):
    xt = inputs.T
    t = targets.astype(jnp.int32)
    hist = _sc_hist(xt, t)
    out = _finish(hist)
    return out[0, 0]
